# baseline (speedup 1.0000x reference)
import functools

import jax
import jax.numpy as jnp
from jax import lax
from jax.experimental import pallas as pl
from jax.experimental.pallas import tpu as pltpu
from jax.experimental.pallas import tpu_sc as plsc

B = 16384
D3 = 256
_GATHER_CHUNK = 128
_NSEG = 2
_BS = B // _NSEG


def _leaky(x):
    return jnp.where(x > 0, x, 0.01 * x)


def _sc_gather_seg(table, idx2d):
    info = plsc.get_sparse_core_info()
    nw = info.num_cores * info.num_subcores
    b_per_w = _BS // nw
    n_chunks = b_per_w // _GATHER_CHUNK

    mesh = plsc.VectorSubcoreMesh(core_axis_name="c", subcore_axis_name="s")

    @functools.partial(
        pl.kernel,
        mesh=mesh,
        out_type=jax.ShapeDtypeStruct((_BS, D3), jnp.float32),
        scratch_types=[
            pltpu.VMEM((n_chunks, _GATHER_CHUNK), jnp.int32),
            pltpu.VMEM((_GATHER_CHUNK, D3), jnp.float32),
            pltpu.VMEM((_GATHER_CHUNK, D3), jnp.float32),
            pltpu.SemaphoreType.DMA,
            pltpu.SemaphoreType.DMA,
            pltpu.SemaphoreType.DMA,
            pltpu.SemaphoreType.DMA,
        ],
    )
    def gather_k(table_hbm, idx_hbm, out_hbm, idx_v, rows_a, rows_b,
                 gsem_a, gsem_b, osem_a, osem_b):
        wid = lax.axis_index("s") * info.num_cores + lax.axis_index("c")
        base = wid * b_per_w
        rows = (rows_a, rows_b)
        gsem = (gsem_a, gsem_b)
        osem = (osem_a, osem_b)
        pltpu.sync_copy(idx_hbm.at[pl.ds(wid * n_chunks, n_chunks)], idx_v)
        out_copies = [None] * n_chunks
        prev = None
        for c in range(n_chunks):
            if c >= 2:
                out_copies[c - 2].wait()
            g = pltpu.async_copy(table_hbm.at[idx_v.at[c]], rows[c % 2],
                                 gsem[c % 2])
            if prev is not None:
                pc, pg = prev
                pg.wait()
                out_copies[pc] = pltpu.async_copy(
                    rows[pc % 2],
                    out_hbm.at[pl.ds(base + pc * _GATHER_CHUNK, _GATHER_CHUNK)],
                    osem[pc % 2])
            prev = (c, g)
        pc, pg = prev
        pg.wait()
        out_copies[pc] = pltpu.async_copy(
            rows[pc % 2],
            out_hbm.at[pl.ds(base + pc * _GATHER_CHUNK, _GATHER_CHUNK)],
            osem[pc % 2])
        if n_chunks >= 2:
            out_copies[n_chunks - 2].wait()
        out_copies[n_chunks - 1].wait()

    return gather_k(table, idx2d)


def _mlp_body(num_ref, W1T_ref, b1_ref, W2T_ref, b2_ref, out_ref):
    x = num_ref[...]
    v = _leaky(jnp.dot(x, W1T_ref[...], preferred_element_type=jnp.float32)
               + b1_ref[...])
    v = _leaky(jnp.dot(v, W2T_ref[...], preferred_element_type=jnp.float32)
               + b2_ref[...])
    out_ref[...] = v


def _tc_mlp(numeric, W1T, b1, W2T, b2, blk=2048):
    def full(shape):
        return pl.BlockSpec(shape, lambda i: (0, 0))

    return pl.pallas_call(
        _mlp_body,
        grid=(B // blk,),
        in_specs=[
            pl.BlockSpec((blk, 3), lambda i: (i, 0)),
            full((3, 64)),
            full((1, 64)),
            full((64, 128)),
            full((1, 128)),
        ],
        out_specs=pl.BlockSpec((blk, 128), lambda i: (i, 0)),
        out_shape=jax.ShapeDtypeStruct((B, 128), jnp.float32),
    )(numeric, W1T, b1, W2T, b2)


def _final_body(v_ref, c1_ref, c2_ref, e3_ref,
                E1_ref, E2_ref, W3vT_ref, W3e1T_ref, W3e2T_ref, W3e3T_ref,
                b3_ref, out_ref):
    blk = v_ref.shape[0]
    acc = jnp.dot(v_ref[...], W3vT_ref[...],
                  preferred_element_type=jnp.float32)
    acc += jnp.dot(e3_ref[...], W3e3T_ref[...],
                   preferred_element_type=jnp.float32)
    oh1 = (c1_ref[...] == lax.broadcasted_iota(jnp.int32, (blk, 4), 1)
           ).astype(jnp.float32)
    e1 = jnp.dot(oh1, E1_ref[...], preferred_element_type=jnp.float32)
    acc += jnp.dot(e1, W3e1T_ref[...], preferred_element_type=jnp.float32)
    oh2 = (c2_ref[...] == lax.broadcasted_iota(jnp.int32, (blk, 5), 1)
           ).astype(jnp.float32)
    e2 = jnp.dot(oh2, E2_ref[...], preferred_element_type=jnp.float32)
    acc += jnp.dot(e2, W3e2T_ref[...], preferred_element_type=jnp.float32)
    acc += b3_ref[...]
    out_ref[...] = _leaky(acc)


def _tc_final_seg(seg, v, c1, c2, e3_seg,
                  E1, E2, W3vT, W3e1T, W3e2T, W3e3T, b3, blk=1024):
    grid = _BS // blk
    off = seg * grid

    def full(shape):
        return pl.BlockSpec(shape, lambda i: (0, 0))

    return pl.pallas_call(
        _final_body,
        grid=(grid,),
        in_specs=[
            pl.BlockSpec((blk, 128), lambda i: (i + off, 0)),
            pl.BlockSpec((blk, 1), lambda i: (i + off, 0)),
            pl.BlockSpec((blk, 1), lambda i: (i + off, 0)),
            pl.BlockSpec((blk, D3), lambda i: (i, 0)),
            full((4, 16)),
            full((5, 25)),
            full((128, 128)),
            full((16, 128)),
            full((25, 128)),
            full((D3, 128)),
            full((1, 128)),
        ],
        out_specs=pl.BlockSpec((blk, 128), lambda i: (i, 0)),
        out_shape=jax.ShapeDtypeStruct((_BS, 128), jnp.float32),
    )(v, c1, c2, e3_seg, E1, E2, W3vT, W3e1T, W3e2T, W3e3T, b3)


def kernel(numeric, cat1, cat2, cat3, W1, b1, W2, b2, E1, E2, E3, W3, b3):
    idx2d = cat3.reshape(B // _GATHER_CHUNK, _GATHER_CHUNK).astype(jnp.int32)
    rows_per_seg = _BS // _GATHER_CHUNK
    e3_segs = [
        _sc_gather_seg(E3, idx2d[s * rows_per_seg:(s + 1) * rows_per_seg])
        for s in range(_NSEG)
    ]

    c1 = cat1.reshape(B, 1).astype(jnp.int32)
    c2 = cat2.reshape(B, 1).astype(jnp.int32)
    v = _tc_mlp(numeric, W1.T, b1.reshape(1, 64), W2.T, b2.reshape(1, 128))

    W3vT = W3[:, 0:128].T
    W3e1T = W3[:, 128:144].T
    W3e2T = W3[:, 144:169].T
    W3e3T = W3[:, 169:425].T
    ys = [
        _tc_final_seg(s, v, c1, c2, e3_segs[s],
                      E1, E2, W3vT, W3e1T, W3e2T, W3e3T, b3.reshape(1, 128))
        for s in range(_NSEG)
    ]
    return jnp.concatenate(ys, axis=0)


# speedup vs baseline: 1.4319x; 1.4319x over previous
import functools

import jax
import jax.numpy as jnp
from jax import lax
from jax.experimental import pallas as pl
from jax.experimental.pallas import tpu as pltpu
from jax.experimental.pallas import tpu_sc as plsc

B = 16384
D3 = 256
_GATHER_CHUNK = 128
_NSEG = 2
_BS = B // _NSEG


def _leaky(x):
    return jnp.where(x > 0, x, 0.01 * x)


def _dot_nt(a, b):
    return lax.dot_general(a, b, (((1,), (1,)), ((), ())),
                           preferred_element_type=jnp.float32)


def _dot_tn(a, b):
    return lax.dot_general(a, b, (((0,), (0,)), ((), ())),
                           preferred_element_type=jnp.float32)


def _sc_gather_seg(table, idx, seg):
    info = plsc.get_sparse_core_info()
    nw = info.num_cores * info.num_subcores
    b_per_w = _BS // nw
    n_chunks = b_per_w // _GATHER_CHUNK

    mesh = plsc.VectorSubcoreMesh(core_axis_name="c", subcore_axis_name="s")

    @functools.partial(
        pl.kernel,
        mesh=mesh,
        out_type=jax.ShapeDtypeStruct((_BS, D3), jnp.float32),
        scratch_types=[
            pltpu.VMEM((1, b_per_w), jnp.int32),
            pltpu.VMEM((_GATHER_CHUNK, D3), jnp.float32),
            pltpu.VMEM((_GATHER_CHUNK, D3), jnp.float32),
            pltpu.SemaphoreType.DMA,
            pltpu.SemaphoreType.DMA,
            pltpu.SemaphoreType.DMA,
            pltpu.SemaphoreType.DMA,
        ],
    )
    def gather_k(table_hbm, idx_hbm, out_hbm, idx_v, rows_a, rows_b,
                 gsem_a, gsem_b, osem_a, osem_b):
        wid = lax.axis_index("s") * info.num_cores + lax.axis_index("c")
        base = wid * b_per_w
        rows = (rows_a, rows_b)
        gsem = (gsem_a, gsem_b)
        osem = (osem_a, osem_b)
        pltpu.sync_copy(
            idx_hbm.at[:, pl.ds(seg * _BS + base, b_per_w)], idx_v)
        out_copies = [None] * n_chunks
        prev = None
        for c in range(n_chunks):
            if c >= 2:
                out_copies[c - 2].wait()
            g = pltpu.async_copy(
                table_hbm.at[idx_v.at[0, pl.ds(c * _GATHER_CHUNK,
                                               _GATHER_CHUNK)]],
                rows[c % 2], gsem[c % 2])
            if prev is not None:
                pc, pg = prev
                pg.wait()
                out_copies[pc] = pltpu.async_copy(
                    rows[pc % 2],
                    out_hbm.at[pl.ds(base + pc * _GATHER_CHUNK, _GATHER_CHUNK)],
                    osem[pc % 2])
            prev = (c, g)
        pc, pg = prev
        pg.wait()
        out_copies[pc] = pltpu.async_copy(
            rows[pc % 2],
            out_hbm.at[pl.ds(base + pc * _GATHER_CHUNK, _GATHER_CHUNK)],
            osem[pc % 2])
        if n_chunks >= 2:
            out_copies[n_chunks - 2].wait()
        out_copies[n_chunks - 1].wait()

    return gather_k(table, idx)


def _mlp_body(num_ref, W1_ref, b1_ref, W2_ref, b2_ref, out_ref):
    x = num_ref[...]
    v = _leaky(_dot_nt(x, W1_ref[...]) + b1_ref[...])
    v = _leaky(_dot_nt(v, W2_ref[...]) + b2_ref[...])
    out_ref[...] = v


def _tc_mlp(numeric, W1, b1, W2, b2, blk=4096):
    def full(shape):
        return pl.BlockSpec(shape, lambda i: tuple(0 for _ in shape))

    return pl.pallas_call(
        _mlp_body,
        grid=(B // blk,),
        in_specs=[
            pl.BlockSpec((blk, 3), lambda i: (i, 0)),
            full((64, 3)),
            full((64,)),
            full((128, 64)),
            full((128,)),
        ],
        out_specs=pl.BlockSpec((blk, 128), lambda i: (i, 0)),
        out_shape=jax.ShapeDtypeStruct((B, 128), jnp.float32),
    )(numeric, W1, b1, W2, b2)


def _final_body(v_ref, c1_ref, c2_ref, e3_ref, E1_ref, E2_ref, W3_ref,
                b3_ref, out_ref):
    blk = v_ref.shape[0]
    acc = _dot_nt(v_ref[...], W3_ref[:, 0:128])
    acc += _dot_nt(e3_ref[...], W3_ref[:, 169:425])
    oh1t = (c1_ref[...] == lax.broadcasted_iota(jnp.int32, (4, blk), 0)
            ).astype(jnp.float32)
    e1 = _dot_tn(oh1t, E1_ref[...])
    acc += _dot_nt(e1, W3_ref[:, 128:144])
    oh2t = (c2_ref[...] == lax.broadcasted_iota(jnp.int32, (5, blk), 0)
            ).astype(jnp.float32)
    e2 = _dot_tn(oh2t, E2_ref[...])
    acc += _dot_nt(e2, W3_ref[:, 144:169])
    acc += b3_ref[...]
    out_ref[...] = _leaky(acc)


def _tc_final_seg(seg, vbuf, c1, c2, e3_seg, E1, E2, W3, b3, blk=2048):
    grid = _BS // blk
    off = seg * grid

    def full(shape):
        return pl.BlockSpec(shape, lambda i: tuple(0 for _ in shape))

    return pl.pallas_call(
        _final_body,
        grid=(grid,),
        in_specs=[
            pl.BlockSpec((blk, 128), lambda i: (i + off, 0)),
            pl.BlockSpec((1, blk), lambda i: (0, i + off)),
            pl.BlockSpec((1, blk), lambda i: (0, i + off)),
            pl.BlockSpec((blk, D3), lambda i: (i, 0)),
            full((4, 16)),
            full((5, 25)),
            full((128, 425)),
            full((128,)),
        ],
        out_specs=pl.BlockSpec((blk, 128), lambda i: (i + off, 0)),
        out_shape=jax.ShapeDtypeStruct((B, 128), jnp.float32),
        input_output_aliases={0: 0},
    )(vbuf, c1, c2, e3_seg, E1, E2, W3, b3)


def kernel(numeric, cat1, cat2, cat3, W1, b1, W2, b2, E1, E2, E3, W3, b3):
    idx = cat3.astype(jnp.int32)
    e3_segs = [_sc_gather_seg(E3, idx, s) for s in range(_NSEG)]

    c1 = cat1.astype(jnp.int32)
    c2 = cat2.astype(jnp.int32)
    v = _tc_mlp(numeric, W1, b1, W2, b2)

    y = v
    for s in range(_NSEG):
        y = _tc_final_seg(s, y, c1, c2, e3_segs[s], E1, E2, W3, b3)
    return y


# speedup vs baseline: 1.5243x; 1.0646x over previous
import functools

import jax
import jax.numpy as jnp
from jax import lax
from jax.experimental import pallas as pl
from jax.experimental.pallas import tpu as pltpu
from jax.experimental.pallas import tpu_sc as plsc

B = 16384
D3 = 256
_GATHER_CHUNK = 128
_NSEG = 2
_BS = B // _NSEG


def _leaky(x):
    return jnp.where(x > 0, x, 0.01 * x)


def _dot_nt(a, b):
    return lax.dot_general(a, b, (((1,), (1,)), ((), ())),
                           preferred_element_type=jnp.float32)


def _dot_tn(a, b):
    return lax.dot_general(a, b, (((0,), (0,)), ((), ())),
                           preferred_element_type=jnp.float32)


def _prep_body(c3_ref, idx_ref, canvas_ref):
    idx_ref[...] = c3_ref[...].reshape(128, 128)
    canvas_ref[...] = jnp.zeros(canvas_ref.shape, canvas_ref.dtype)


def _tc_prep(cat3):
    return pl.pallas_call(
        _prep_body,
        grid=(1,),
        in_specs=[pl.BlockSpec((1, B), lambda i: (0, 0))],
        out_specs=[
            pl.BlockSpec((128, 128), lambda i: (0, 0)),
            pl.BlockSpec((8, 128), lambda i: (0, 0)),
        ],
        out_shape=[
            jax.ShapeDtypeStruct((128, 128), jnp.int32),
            jax.ShapeDtypeStruct((B, 128), jnp.float32),
        ],
    )(cat3)


def _sc_gather_seg(table, idx2d, seg):
    info = plsc.get_sparse_core_info()
    nw = info.num_cores * info.num_subcores
    b_per_w = _BS // nw
    n_chunks = b_per_w // _GATHER_CHUNK
    rows_per_w = b_per_w // _GATHER_CHUNK

    mesh = plsc.VectorSubcoreMesh(core_axis_name="c", subcore_axis_name="s")

    @functools.partial(
        pl.kernel,
        mesh=mesh,
        out_type=jax.ShapeDtypeStruct((_BS, D3), jnp.float32),
        scratch_types=[
            pltpu.VMEM((8, _GATHER_CHUNK), jnp.int32),
            pltpu.VMEM((_GATHER_CHUNK, D3), jnp.float32),
            pltpu.VMEM((_GATHER_CHUNK, D3), jnp.float32),
            pltpu.SemaphoreType.DMA,
            pltpu.SemaphoreType.DMA,
            pltpu.SemaphoreType.DMA,
            pltpu.SemaphoreType.DMA,
        ],
    )
    def gather_k(table_hbm, idx_hbm, out_hbm, idx_v, rows_a, rows_b,
                 gsem_a, gsem_b, osem_a, osem_b):
        wid = lax.axis_index("s") * info.num_cores + lax.axis_index("c")
        base = wid * b_per_w
        idx_row0 = (seg * _BS + base) // _GATHER_CHUNK
        slab0 = pl.multiple_of((idx_row0 // 8) * 8, 8)
        inner = idx_row0 - (idx_row0 // 8) * 8
        rows = (rows_a, rows_b)
        gsem = (gsem_a, gsem_b)
        osem = (osem_a, osem_b)
        pltpu.sync_copy(idx_hbm.at[pl.ds(slab0, 8)], idx_v)
        out_copies = [None] * n_chunks
        prev = None
        for c in range(n_chunks):
            if c >= 2:
                out_copies[c - 2].wait()
            g = pltpu.async_copy(table_hbm.at[idx_v.at[inner + c]],
                                 rows[c % 2], gsem[c % 2])
            if prev is not None:
                pc, pg = prev
                pg.wait()
                out_copies[pc] = pltpu.async_copy(
                    rows[pc % 2],
                    out_hbm.at[pl.ds(base + pc * _GATHER_CHUNK, _GATHER_CHUNK)],
                    osem[pc % 2])
            prev = (c, g)
        pc, pg = prev
        pg.wait()
        out_copies[pc] = pltpu.async_copy(
            rows[pc % 2],
            out_hbm.at[pl.ds(base + pc * _GATHER_CHUNK, _GATHER_CHUNK)],
            osem[pc % 2])
        if n_chunks >= 2:
            out_copies[n_chunks - 2].wait()
        out_copies[n_chunks - 1].wait()

    return gather_k(table, idx2d)


def _final_body(canvas_ref, num_ref, c1_ref, c2_ref, e3_ref,
                W1_ref, b1_ref, W2_ref, b2_ref, E1_ref, E2_ref, W3_ref,
                b3_ref, out_ref):
    blk = num_ref.shape[0]
    x = num_ref[...]
    v = _leaky(_dot_nt(x, W1_ref[...]) + b1_ref[...])
    v = _leaky(_dot_nt(v, W2_ref[...]) + b2_ref[...])
    acc = _dot_nt(v, W3_ref[:, 0:128])
    acc += _dot_nt(e3_ref[...], W3_ref[:, 169:425])
    oh1t = (c1_ref[...] == lax.broadcasted_iota(jnp.int32, (4, blk), 0)
            ).astype(jnp.float32)
    e1 = _dot_tn(oh1t, E1_ref[...])
    acc += _dot_nt(e1, W3_ref[:, 128:144])
    oh2t = (c2_ref[...] == lax.broadcasted_iota(jnp.int32, (5, blk), 0)
            ).astype(jnp.float32)
    e2 = _dot_tn(oh2t, E2_ref[...])
    acc += _dot_nt(e2, W3_ref[:, 144:169])
    acc += b3_ref[...]
    out_ref[...] = _leaky(acc)


def _tc_final_seg(seg, canvas, numeric, c1, c2, e3_seg,
                  W1, b1, W2, b2, E1, E2, W3, b3, blk=2048):
    grid = _BS // blk
    off = seg * grid

    def full(shape):
        return pl.BlockSpec(shape, lambda i: tuple(0 for _ in shape))

    return pl.pallas_call(
        _final_body,
        grid=(grid,),
        in_specs=[
            pl.BlockSpec((blk, 128), lambda i: (i + off, 0)),
            pl.BlockSpec((blk, 3), lambda i: (i + off, 0)),
            pl.BlockSpec((1, blk), lambda i: (0, i + off)),
            pl.BlockSpec((1, blk), lambda i: (0, i + off)),
            pl.BlockSpec((blk, D3), lambda i: (i, 0)),
            full((64, 3)),
            full((64,)),
            full((128, 64)),
            full((128,)),
            full((4, 16)),
            full((5, 25)),
            full((128, 425)),
            full((128,)),
        ],
        out_specs=pl.BlockSpec((blk, 128), lambda i: (i + off, 0)),
        out_shape=jax.ShapeDtypeStruct((B, 128), jnp.float32),
        input_output_aliases={0: 0},
    )(canvas, numeric, c1, c2, e3_seg, W1, b1, W2, b2, E1, E2, W3, b3)


def kernel(numeric, cat1, cat2, cat3, W1, b1, W2, b2, E1, E2, E3, W3, b3):
    idx2d, canvas = _tc_prep(cat3.astype(jnp.int32))
    e3_segs = [_sc_gather_seg(E3, idx2d, s) for s in range(_NSEG)]

    c1 = cat1.astype(jnp.int32)
    c2 = cat2.astype(jnp.int32)
    y = canvas
    for s in range(_NSEG):
        y = _tc_final_seg(s, y, numeric, c1, c2, e3_segs[s],
                          W1, b1, W2, b2, E1, E2, W3, b3)
    return y
